# Initial kernel scaffold; baseline (speedup 1.0000x reference)
import jax
import jax.numpy as jnp
from jax.experimental import pallas as pl


def kernel(x, edge_index, W1_l, b1_l, W1_r, W2_l, b2_l, W2_r):
    raise NotImplementedError("write your pallas kernel here")


# speedup vs baseline: 6.0912x; 6.0912x over previous
import functools

import jax
import jax.numpy as jnp
from jax import lax
from jax.experimental import pallas as pl
from jax.experimental.pallas import tpu as pltpu
from jax.experimental.pallas import tpu_sc as plsc

N = 10000
E = 320000
D = 128
NC, NS = 2, 16
NW = NC * NS
EPT = E // NW
CHK = 125
NCH = EPT // CHK
GRP = 8
NG = NCH // GRP
ZB = 624
ZT = N - ZB * NS
ZC = 48
NZ = ZB // ZC


@functools.lru_cache(maxsize=None)
def _make_segsum(tag):
    mesh = plsc.VectorSubcoreMesh(
        core_axis_name="c", subcore_axis_name="s", num_cores=NC, num_subcores=NS
    )
    scratch = [
        pltpu.VMEM((ZC, D), jnp.float32),
        pltpu.VMEM((GRP, CHK), jnp.int32),
        pltpu.VMEM((GRP, CHK), jnp.int32),
        pltpu.VMEM((CHK, D), jnp.float32),
        pltpu.VMEM_SHARED((N, D), jnp.float32),
        pltpu.SemaphoreType.DMA,
    ]

    @functools.partial(pl.kernel,
                       out_type=jax.ShapeDtypeStruct((NC * N, D), jnp.float32),
                       mesh=mesh, scratch_types=scratch,
                       name=f"segsum_{tag}")
    def seg(h_hbm, src_hbm, dst_hbm, zero_hbm, agg_out,
            stg_v, srcs_v, dsts_v, rows_v, acc_sh, sem):
        cid = lax.axis_index("c")
        sid = lax.axis_index("s")
        wid = cid * NS + sid
        base = pl.multiple_of(sid * ZB, 8)
        obase = pl.multiple_of(cid * N + sid * ZB, 8)
        otail = pl.multiple_of(cid * N + ZB * NS, 8)

        pltpu.sync_copy(zero_hbm, stg_v)
        for j in range(NZ):
            pltpu.sync_copy(stg_v, acc_sh.at[pl.ds(base + j * ZC, ZC)])

        @pl.when(sid == 0)
        def _():
            pltpu.sync_copy(stg_v.at[pl.ds(0, ZT)],
                            acc_sh.at[pl.ds(ZB * NS, ZT)])

        plsc.subcore_barrier()

        for c in range(NCH):
            g, r = divmod(c, GRP)
            if r == 0:
                pltpu.sync_copy(src_hbm.at[wid, pl.ds(g * GRP, GRP)], srcs_v)
                pltpu.sync_copy(dst_hbm.at[wid, pl.ds(g * GRP, GRP)], dsts_v)
            pltpu.async_copy(h_hbm.at[srcs_v.at[r]], rows_v, sem).wait()
            pltpu.sync_copy(rows_v, acc_sh.at[dsts_v.at[r]], add=True)

        plsc.subcore_barrier()

        for j in range(NZ):
            pltpu.sync_copy(acc_sh.at[pl.ds(base + j * ZC, ZC)], stg_v)
            pltpu.sync_copy(stg_v, agg_out.at[pl.ds(obase + j * ZC, ZC)])

        @pl.when(sid == 0)
        def _():
            pltpu.sync_copy(acc_sh.at[pl.ds(ZB * NS, ZT)],
                            stg_v.at[pl.ds(0, ZT)])
            pltpu.sync_copy(stg_v.at[pl.ds(0, ZT)],
                            agg_out.at[pl.ds(otail, ZT)])

    return seg


R = 1000


def _dense_body(final, aggp, degp, h, wl, b, wr, out):
    agg = aggp[0] + aggp[1]
    deg = degp[0, :, 0:1] + degp[1, :, 0:1]
    mean = agg * (1.0 / jnp.maximum(deg, 1.0))
    dn = (((1,), (1,)), ((), ()))
    o = (lax.dot_general(mean, wl[...], dn, preferred_element_type=jnp.float32)
         + lax.dot_general(h[...], wr[...], dn,
                           preferred_element_type=jnp.float32)
         + b[...])
    nrm = jnp.sqrt(jnp.sum(o * o, axis=1, keepdims=True))
    o = o / jnp.maximum(nrm, 1e-12)
    if final:
        s = o - jnp.max(o, axis=1, keepdims=True)
        out[...] = s - jnp.log(jnp.sum(jnp.exp(s), axis=1, keepdims=True))
    else:
        out[...] = jnp.maximum(o, 0.0)


def _make_dense(final):
    return pl.pallas_call(
        functools.partial(_dense_body, final),
        grid=(N // R,),
        in_specs=[
            pl.BlockSpec((NC, R, D), lambda i: (0, i, 0)),
            pl.BlockSpec((NC, R, D), lambda i: (0, i, 0)),
            pl.BlockSpec((R, D), lambda i: (i, 0)),
            pl.BlockSpec((D, D), lambda i: (0, 0)),
            pl.BlockSpec((1, D), lambda i: (0, 0)),
            pl.BlockSpec((D, D), lambda i: (0, 0)),
        ],
        out_specs=pl.BlockSpec((R, D), lambda i: (i, 0)),
        out_shape=jax.ShapeDtypeStruct((N, D), jnp.float32),
    )


_dense_relu = _make_dense(False)
_dense_lsm = _make_dense(True)


def kernel(x, edge_index, W1_l, b1_l, W1_r, W2_l, b2_l, W2_r):
    src4 = edge_index[0].reshape(NW, NCH, CHK)
    dst4 = edge_index[1].reshape(NW, NCH, CHK)
    zero = jnp.zeros((ZC, D), jnp.float32)
    ones_tbl = jnp.ones((N, D), jnp.float32)
    degp = _make_segsum("deg")(ones_tbl, src4, dst4, zero).reshape(NC, N, D)
    x_ser, _ = lax.optimization_barrier((x, degp))
    aggp1 = _make_segsum("l1")(x_ser, src4, dst4, zero).reshape(NC, N, D)
    h1 = _dense_relu(aggp1, degp, x, W1_l, b1_l.reshape(1, D), W1_r)
    aggp2 = _make_segsum("l2")(h1, src4, dst4, zero).reshape(NC, N, D)
    out = _dense_lsm(aggp2, degp, h1, W2_l, b2_l.reshape(1, D), W2_r)
    return out


# speedup vs baseline: 8.2554x; 1.3553x over previous
import functools

import jax
import jax.numpy as jnp
from jax import lax
from jax.experimental import pallas as pl
from jax.experimental.pallas import tpu as pltpu
from jax.experimental.pallas import tpu_sc as plsc

N = 10000
E = 320000
D = 128
NC, NS = 2, 16
NW = NC * NS
EPT = E // NW
CHK = 125
NCH = EPT // CHK
GRP = 8
NG = NCH // GRP
ZB = 624
ZT = N - ZB * NS
ZC = 48
NZ = ZB // ZC


@functools.lru_cache(maxsize=None)
def _make_segsum(tag):
    mesh = plsc.VectorSubcoreMesh(
        core_axis_name="c", subcore_axis_name="s", num_cores=NC, num_subcores=NS
    )
    scratch = [
        pltpu.VMEM((ZC, D), jnp.float32),
        pltpu.VMEM((GRP, CHK), jnp.int32),
        pltpu.VMEM((GRP, CHK), jnp.int32),
        pltpu.VMEM((CHK, D), jnp.float32),
        pltpu.VMEM((CHK, D), jnp.float32),
        pltpu.VMEM_SHARED((N, D), jnp.float32),
        pltpu.SemaphoreType.DMA,
        pltpu.SemaphoreType.DMA,
    ]

    @functools.partial(pl.kernel,
                       out_type=jax.ShapeDtypeStruct((NC * N, D), jnp.float32),
                       mesh=mesh, scratch_types=scratch,
                       name=f"segsum_{tag}")
    def seg(h_hbm, src_hbm, dst_hbm, zero_hbm, agg_out,
            stg_v, srcs_v, dsts_v, rows0_v, rows1_v, acc_sh, sem0, sem1):
        rows = (rows0_v, rows1_v)
        sems = (sem0, sem1)
        cid = lax.axis_index("c")
        sid = lax.axis_index("s")
        wid = cid * NS + sid
        base = pl.multiple_of(sid * ZB, 8)
        obase = pl.multiple_of(cid * N + sid * ZB, 8)
        otail = pl.multiple_of(cid * N + ZB * NS, 8)

        pltpu.sync_copy(zero_hbm, stg_v)
        for j in range(NZ):
            pltpu.sync_copy(stg_v, acc_sh.at[pl.ds(base + j * ZC, ZC)])

        @pl.when(sid == 0)
        def _():
            pltpu.sync_copy(stg_v.at[pl.ds(0, ZT)],
                            acc_sh.at[pl.ds(ZB * NS, ZT)])

        plsc.subcore_barrier()

        for g in range(NG):
            pltpu.sync_copy(src_hbm.at[wid, pl.ds(g * GRP, GRP)], srcs_v)
            pltpu.sync_copy(dst_hbm.at[wid, pl.ds(g * GRP, GRP)], dsts_v)
            pending = pltpu.async_copy(h_hbm.at[srcs_v.at[0]],
                                       rows[0], sems[0])
            for r in range(GRP):
                if r + 1 < GRP:
                    nxt = pltpu.async_copy(
                        h_hbm.at[srcs_v.at[r + 1]], rows[(r + 1) % 2],
                        sems[(r + 1) % 2])
                pending.wait()
                pltpu.sync_copy(rows[r % 2], acc_sh.at[dsts_v.at[r]],
                                add=True)
                if r + 1 < GRP:
                    pending = nxt

        plsc.subcore_barrier()

        for j in range(NZ):
            pltpu.sync_copy(acc_sh.at[pl.ds(base + j * ZC, ZC)], stg_v)
            pltpu.sync_copy(stg_v, agg_out.at[pl.ds(obase + j * ZC, ZC)])

        @pl.when(sid == 0)
        def _():
            pltpu.sync_copy(acc_sh.at[pl.ds(ZB * NS, ZT)],
                            stg_v.at[pl.ds(0, ZT)])
            pltpu.sync_copy(stg_v.at[pl.ds(0, ZT)],
                            agg_out.at[pl.ds(otail, ZT)])

    return seg


R = 1000


def _dense_body(final, aggp, degp, h, wl, b, wr, out):
    agg = aggp[0] + aggp[1]
    deg = degp[0, :, 0:1] + degp[1, :, 0:1]
    mean = agg * (1.0 / jnp.maximum(deg, 1.0))
    dn = (((1,), (1,)), ((), ()))
    o = (lax.dot_general(mean, wl[...], dn, preferred_element_type=jnp.float32)
         + lax.dot_general(h[...], wr[...], dn,
                           preferred_element_type=jnp.float32)
         + b[...])
    nrm = jnp.sqrt(jnp.sum(o * o, axis=1, keepdims=True))
    o = o / jnp.maximum(nrm, 1e-12)
    if final:
        s = o - jnp.max(o, axis=1, keepdims=True)
        out[...] = s - jnp.log(jnp.sum(jnp.exp(s), axis=1, keepdims=True))
    else:
        out[...] = jnp.maximum(o, 0.0)


def _make_dense(final):
    return pl.pallas_call(
        functools.partial(_dense_body, final),
        grid=(N // R,),
        in_specs=[
            pl.BlockSpec((NC, R, D), lambda i: (0, i, 0)),
            pl.BlockSpec((NC, R, D), lambda i: (0, i, 0)),
            pl.BlockSpec((R, D), lambda i: (i, 0)),
            pl.BlockSpec((D, D), lambda i: (0, 0)),
            pl.BlockSpec((1, D), lambda i: (0, 0)),
            pl.BlockSpec((D, D), lambda i: (0, 0)),
        ],
        out_specs=pl.BlockSpec((R, D), lambda i: (i, 0)),
        out_shape=jax.ShapeDtypeStruct((N, D), jnp.float32),
    )


_dense_relu = _make_dense(False)
_dense_lsm = _make_dense(True)


def kernel(x, edge_index, W1_l, b1_l, W1_r, W2_l, b2_l, W2_r):
    src4 = edge_index[0].reshape(NW, NCH, CHK)
    dst4 = edge_index[1].reshape(NW, NCH, CHK)
    zero = jnp.zeros((ZC, D), jnp.float32)
    ones_tbl = jnp.ones((N, D), jnp.float32)
    degp = _make_segsum("deg")(ones_tbl, src4, dst4, zero).reshape(NC, N, D)
    x_ser, _ = lax.optimization_barrier((x, degp))
    aggp1 = _make_segsum("l1")(x_ser, src4, dst4, zero).reshape(NC, N, D)
    h1 = _dense_relu(aggp1, degp, x, W1_l, b1_l.reshape(1, D), W1_r)
    aggp2 = _make_segsum("l2")(h1, src4, dst4, zero).reshape(NC, N, D)
    out = _dense_lsm(aggp2, degp, h1, W2_l, b2_l.reshape(1, D), W2_r)
    return out
